# baseline (speedup 1.0000x reference)
import functools

import jax
import jax.numpy as jnp
import numpy as np
from jax import lax
from jax.experimental import pallas as pl
from jax.experimental.pallas import tpu as pltpu

_B = 8
_NPTS = 4096
_NCLS = 13


def _gather_j(x, idx):
    bidx = jnp.arange(x.shape[0]).reshape((-1,) + (1,) * (idx.ndim - 1))
    return x[bidx, idx]


def _fps_body(m, n, px, py, pz, o_ref):
    x = px[...]
    y = py[...]
    z = pz[...]
    iota = jax.lax.broadcasted_iota(jnp.int32, (_B, n), 1)
    iom = jax.lax.broadcasted_iota(jnp.int32, (_B, m), 1)

    def step(t, carry):
        mind, last, acc = carry
        acc = acc + jnp.where(iom == t, 1, 0) * last
        sel = iota == last
        lx = jnp.sum(jnp.where(sel, x, 0.0), axis=1, keepdims=True)
        ly = jnp.sum(jnp.where(sel, y, 0.0), axis=1, keepdims=True)
        lz = jnp.sum(jnp.where(sel, z, 0.0), axis=1, keepdims=True)
        dx = x - lx
        dy = y - ly
        dz = z - lz
        d = dx * dx + dy * dy
        d = d + dz * dz
        mind = jnp.minimum(mind, d)
        mx = jnp.max(mind, axis=1, keepdims=True)
        cand = jnp.where(mind == mx, iota, n)
        nxt = jnp.min(cand, axis=1, keepdims=True)
        return mind, nxt, acc

    o_ref[...] = jnp.zeros((_B, m), jnp.int32)
    init = (jnp.full((_B, n), 1e10, jnp.float32),
            jnp.zeros((_B, 1), jnp.int32),
            o_ref[...])
    _, _, acc = jax.lax.fori_loop(0, m, step, init)
    o_ref[...] = acc


def _fps_pallas(pos, m):
    n = pos.shape[1]
    planes = jnp.transpose(pos, (2, 0, 1))
    px, py, pz = planes[0], planes[1], planes[2]
    body = functools.partial(_fps_body, m, n)
    return pl.pallas_call(
        body,
        grid=(1,),
        in_specs=[pl.BlockSpec((_B, n), lambda i: (0, 0))] * 3,
        out_specs=pl.BlockSpec((_B, m), lambda i: (0, 0)),
        out_shape=jax.ShapeDtypeStruct((_B, m), jnp.int32),
    )(px, py, pz)


def _knn3_body(q_ref, ct_ref, v_ref, o_ref):
    q = q_ref[0]
    cT = ct_ref[0]
    nc = cT.shape[1]
    bq = q.shape[0]
    qn = jnp.sum(q * q, axis=1, keepdims=True)
    cn = jnp.sum(cT * cT, axis=0, keepdims=True)
    d2 = qn - 2.0 * jnp.dot(q, cT, preferred_element_type=jnp.float32) + cn
    iota = jax.lax.broadcasted_iota(jnp.int32, (bq, nc), 1)
    wmat = jnp.zeros((bq, nc), jnp.float32)
    wsum = jnp.zeros((bq, 1), jnp.float32)
    for _ in range(3):
        mv = jnp.min(d2, axis=1, keepdims=True)
        sel = d2 == mv
        first = jnp.min(jnp.where(sel, iota, nc), axis=1, keepdims=True)
        onehot = iota == first
        w = 1.0 / jnp.maximum(mv, 1e-10)
        wmat = wmat + jnp.where(onehot, w, 0.0)
        wsum = wsum + w
        d2 = jnp.where(onehot, jnp.inf, d2)
    wmat = wmat / wsum
    o_ref[0] = jnp.dot(wmat, v_ref[0], preferred_element_type=jnp.float32)


def _knn3_pallas(qpos, cpos, vals, bq):
    Bb, nq, _ = qpos.shape
    nc = cpos.shape[1]
    C = vals.shape[2]
    cT = jnp.transpose(cpos, (0, 2, 1))
    return pl.pallas_call(
        _knn3_body,
        grid=(Bb, nq // bq),
        in_specs=[
            pl.BlockSpec((1, bq, 3), lambda b, i: (b, i, 0)),
            pl.BlockSpec((1, 3, nc), lambda b, i: (b, 0, 0)),
            pl.BlockSpec((1, nc, C), lambda b, i: (b, 0, 0)),
        ],
        out_specs=pl.BlockSpec((1, bq, C), lambda b, i: (b, i, 0)),
        out_shape=jax.ShapeDtypeStruct((Bb, nq, C), jnp.float32),
    )(qpos, cT, vals)


def _mlp_stack_body(nlin, relu_flags, softmax, maxpool, x_ref, *refs):
    o_ref = refs[-1]
    h = x_ref[0] if x_ref.shape[0] == 1 and len(x_ref.shape) == 3 else x_ref[...]
    for li in range(nlin):
        W = refs[2 * li][...]
        b = refs[2 * li + 1][...]
        h = jnp.dot(h, W, preferred_element_type=jnp.float32) + b
        if relu_flags[li]:
            h = jnp.maximum(h, 0.0)
    if softmax:
        mx = jnp.max(h, axis=-1, keepdims=True)
        h = (h - mx) - jnp.log(jnp.sum(jnp.exp(h - mx), axis=-1, keepdims=True))
    if maxpool:
        h = jnp.max(h, axis=0, keepdims=True)
    if len(o_ref.shape) == 3:
        o_ref[0] = h
    else:
        o_ref[...] = h


def _mlp_rows(x, layers, relu_flags=None, softmax=False, bm=2048):
    n, c = x.shape
    nlin = len(layers)
    if relu_flags is None:
        relu_flags = [True] * nlin
    ws = []
    for (W, b) in layers:
        ws.append(W)
        ws.append(b.reshape(1, -1))
    cout = layers[-1][0].shape[1]
    body = functools.partial(_mlp_stack_body, nlin, tuple(relu_flags),
                             softmax, False)
    return pl.pallas_call(
        body,
        grid=(n // bm,),
        in_specs=[pl.BlockSpec((bm, c), lambda i: (i, 0))]
        + [pl.BlockSpec(w.shape, lambda i: (0, 0)) for w in ws],
        out_specs=pl.BlockSpec((bm, cout), lambda i: (i, 0)),
        out_shape=jax.ShapeDtypeStruct((n, cout), jnp.float32),
    )(x, *ws)


def _mlp_batched(x, layers, maxpool=False):
    Bb, n, c = x.shape
    nlin = len(layers)
    ws = []
    for (W, b) in layers:
        ws.append(W)
        ws.append(b.reshape(1, -1))
    cout = layers[-1][0].shape[1]
    nout = 1 if maxpool else n
    body = functools.partial(_mlp_stack_body, nlin, tuple([True] * nlin),
                             False, maxpool)
    out = pl.pallas_call(
        body,
        grid=(Bb,),
        in_specs=[pl.BlockSpec((1, n, c), lambda b: (b, 0, 0))]
        + [pl.BlockSpec(w.shape, lambda b: (0, 0)) for w in ws],
        out_specs=pl.BlockSpec((1, nout, cout), lambda b: (b, 0, 0)),
        out_shape=jax.ShapeDtypeStruct((Bb, nout, cout), jnp.float32),
    )(x, *ws)
    return out[:, 0, :] if maxpool else out


def _ball_level(pos, centers, specs):
    d2 = jnp.sum((centers[:, :, None, :] - pos[:, None, :, :]) ** 2, axis=-1)
    kmax = max(K for (_, K) in specs)
    negd, sidx = jax.lax.top_k(-d2, kmax)
    sd2 = -negd
    outs = []
    for (r, K) in specs:
        idxK = sidx[..., :K]
        valid = sd2[..., :K] <= r * r
        outs.append(jnp.where(valid, idxK, idxK[..., :1]))
    return outs


def _g1_body(K, bm, r_ref, w1, b1, w2, b2, w3, b3, o_ref):
    h = r_ref[...]
    h = jnp.maximum(jnp.dot(h, w1[...], preferred_element_type=jnp.float32) + b1[...], 0.0)
    h = jnp.maximum(jnp.dot(h, w2[...], preferred_element_type=jnp.float32) + b2[...], 0.0)
    h = jnp.maximum(jnp.dot(h, w3[...], preferred_element_type=jnp.float32) + b3[...], 0.0)
    for i in range(bm):
        o_ref[i:i + 1, :] = jnp.max(h[i * K:(i + 1) * K, :], axis=0,
                                    keepdims=True)


def _grouped_mlp_rel(rel2d, K, bm, layers):
    R = rel2d.shape[0]
    c3 = layers[2][0].shape[1]
    nctr = R // K
    ws = []
    for (W, b) in layers:
        ws.append(W)
        ws.append(b.reshape(1, -1))
    body = functools.partial(_g1_body, K, bm)
    return pl.pallas_call(
        body,
        grid=(R // (bm * K),),
        in_specs=[pl.BlockSpec((bm * K, 3), lambda g: (g, 0))]
        + [pl.BlockSpec(w.shape, lambda g: (0, 0)) for w in ws],
        out_specs=pl.BlockSpec((bm, c3), lambda g: (g, 0)),
        out_shape=jax.ShapeDtypeStruct((nctr, c3), jnp.float32),
    )(rel2d, *ws)


def _g2_body(K, bm, n, a_ref, bc_ref, idx_ref, w2, b2, w3, b3, o_ref):
    A = a_ref[0]
    idxv = idx_ref[...]
    lane_n = jax.lax.broadcasted_iota(jnp.int32, (bm * K, n), 1)
    oh = jnp.where(idxv == lane_n, 1.0, 0.0)
    G = jnp.dot(oh, A, preferred_element_type=jnp.float32)
    gid = jax.lax.broadcasted_iota(jnp.int32, (bm * K, 1), 0) // K
    lane_b = jax.lax.broadcasted_iota(jnp.int32, (bm * K, bm), 1)
    ohg = jnp.where(gid == lane_b, 1.0, 0.0)
    Brep = jnp.dot(ohg, bc_ref[0], preferred_element_type=jnp.float32)
    h = jnp.maximum(G - Brep, 0.0)
    h = jnp.maximum(jnp.dot(h, w2[...], preferred_element_type=jnp.float32) + b2[...], 0.0)
    h = jnp.maximum(jnp.dot(h, w3[...], preferred_element_type=jnp.float32) + b3[...], 0.0)
    for i in range(bm):
        o_ref[0, i:i + 1, :] = jnp.max(h[i * K:(i + 1) * K, :], axis=0,
                                       keepdims=True)


def _grouped_mlp_gather(A, Bc2d, idxcol, K, bm, layers23):
    Bb, n, c1 = A.shape
    m = Bc2d.shape[0] // Bb
    c3 = layers23[1][0].shape[1]
    gpb = m // bm
    ws = []
    for (W, b) in layers23:
        ws.append(W)
        ws.append(b.reshape(1, -1))
    body = functools.partial(_g2_body, K, bm, n)
    ngrp = Bb * gpb
    bc3 = Bc2d.reshape(ngrp, bm, c1)
    out = pl.pallas_call(
        body,
        grid=(Bb, gpb),
        in_specs=[
            pl.BlockSpec((1, n, c1), lambda b, g: (b, 0, 0)),
            pl.BlockSpec((1, bm, c1), lambda b, g, _g=gpb: (b * _g + g, 0, 0)),
            pl.BlockSpec((bm * K, 1), lambda b, g, _g=gpb: (b * _g + g, 0)),
        ]
        + [pl.BlockSpec(w.shape, lambda b, g: (0, 0)) for w in ws],
        out_specs=pl.BlockSpec((1, bm, c3), lambda b, g, _g=gpb: (b * _g + g, 0, 0)),
        out_shape=jax.ShapeDtypeStruct((ngrp, bm, c3), jnp.float32),
    )(A, bc3, idxcol, *ws)
    return out.reshape(Bb * m, c3)


def _sa_msg1(pos, m, radii, Ks, mlps, bms):
    cidx = _fps_pallas(pos, m)
    centers = _gather_j(pos, cidx)
    idxs = _ball_level(pos, centers, list(zip(radii, Ks)))
    outs = []
    for idx, ps, K, bm in zip(idxs, mlps, Ks, bms):
        rel = _gather_j(pos, idx) - centers[:, :, None, :]
        rel2d = rel.reshape(_B * m * K, 3)
        o = _grouped_mlp_rel(rel2d, K, bm, ps)
        outs.append(o.reshape(_B, m, -1))
    return jnp.concatenate(outs, axis=-1), centers


def _sa_msg2(x, pos, m, radii, Ks, mlps, bms):
    cidx = _fps_pallas(pos, m)
    centers = _gather_j(pos, cidx)
    idxs = _ball_level(pos, centers, list(zip(radii, Ks)))
    n = pos.shape[1]
    xp = jnp.concatenate([x, pos], axis=-1).reshape(_B * n, -1)
    cflat = centers.reshape(_B * m, 3)
    outs = []
    for idx, ps, K, bm in zip(idxs, mlps, Ks, bms):
        (W1, b1), l2, l3 = ps
        A = _mlp_rows(xp, [(W1, b1)], relu_flags=[False],
                      bm=_B * n // 8).reshape(_B, n, -1)
        W1p = W1[-3:]
        Bc2d = _mlp_rows(cflat, [(W1p, jnp.zeros((W1.shape[1],), jnp.float32))],
                         relu_flags=[False], bm=_B * m // 8)
        idxcol = idx.reshape(_B * m * K, 1)
        o = _grouped_mlp_gather(A, Bc2d, idxcol, K, bm, [l2, l3])
        outs.append(o.reshape(_B, m, -1))
    return jnp.concatenate(outs, axis=-1), centers


def kernel(pos, batch, params):
    del batch
    pos3 = pos.reshape(_B, _NPTS, 3)
    x1, p1 = _sa_msg1(pos3, 512, (0.1, 0.2, 0.4), (16, 32, 128),
                      params['sa1'], bms=(64, 32, 16))
    x2, p2 = _sa_msg2(x1, p1, 128, (0.2, 0.4, 0.8), (32, 64, 128),
                      params['sa2'], bms=(16, 8, 4))
    g = _mlp_batched(jnp.concatenate([x2, p2], axis=-1), params['sa3'],
                     maxpool=True)
    gb = jnp.broadcast_to(g[:, None, :], (_B, 128, g.shape[1]))
    h3 = _mlp_batched(jnp.concatenate([gb, x2], axis=-1), params['fp3'])
    xi2 = _knn3_pallas(p1, p2, h3, bq=512)
    h2 = _mlp_batched(jnp.concatenate([xi2, x1], axis=-1), params['fp2'])
    xi1 = _knn3_pallas(pos3, p1, h2, bq=1024)
    head = list(params['fp1']) + [params['lin1'], params['lin2'], params['lin3']]
    out = _mlp_rows(xi1.reshape(_B * _NPTS, 128), head,
                    relu_flags=[True, True, True, True, False, False],
                    softmax=True, bm=2048)
    return out


# speedup vs baseline: 1.0934x; 1.0934x over previous
import functools

import jax
import jax.numpy as jnp
import numpy as np
from jax import lax
from jax.experimental import pallas as pl
from jax.experimental.pallas import tpu as pltpu

_B = 8
_NPTS = 4096
_NCLS = 13


def _gather_j(x, idx):
    bidx = jnp.arange(x.shape[0]).reshape((-1,) + (1,) * (idx.ndim - 1))
    return x[bidx, idx]


def _fps_body(m, n, px, py, pz, o_ref):
    x = px[...]
    y = py[...]
    z = pz[...]
    iota = jax.lax.broadcasted_iota(jnp.int32, (_B, n), 1)
    iom = jax.lax.broadcasted_iota(jnp.int32, (_B, m), 1)

    def step(t, carry):
        mind, last, acc = carry
        acc = acc + jnp.where(iom == t, 1, 0) * last
        sel = iota == last
        lx = jnp.sum(jnp.where(sel, x, 0.0), axis=1, keepdims=True)
        ly = jnp.sum(jnp.where(sel, y, 0.0), axis=1, keepdims=True)
        lz = jnp.sum(jnp.where(sel, z, 0.0), axis=1, keepdims=True)
        dx = x - lx
        dy = y - ly
        dz = z - lz
        d = dx * dx + dy * dy
        d = d + dz * dz
        mind = jnp.minimum(mind, d)
        mx = jnp.max(mind, axis=1, keepdims=True)
        cand = jnp.where(mind == mx, iota, n)
        nxt = jnp.min(cand, axis=1, keepdims=True)
        return mind, nxt, acc

    o_ref[...] = jnp.zeros((_B, m), jnp.int32)
    init = (jnp.full((_B, n), 1e10, jnp.float32),
            jnp.zeros((_B, 1), jnp.int32),
            o_ref[...])
    _, _, acc = jax.lax.fori_loop(0, m, step, init)
    o_ref[...] = acc


def _fps_pallas(pos, m):
    n = pos.shape[1]
    planes = jnp.transpose(pos, (2, 0, 1))
    px, py, pz = planes[0], planes[1], planes[2]
    body = functools.partial(_fps_body, m, n)
    return pl.pallas_call(
        body,
        grid=(1,),
        in_specs=[pl.BlockSpec((_B, n), lambda i: (0, 0))] * 3,
        out_specs=pl.BlockSpec((_B, m), lambda i: (0, 0)),
        out_shape=jax.ShapeDtypeStruct((_B, m), jnp.int32),
    )(px, py, pz)


def _knn3_body(q_ref, ct_ref, v_ref, o_ref):
    q = q_ref[0]
    cT = ct_ref[0]
    nc = cT.shape[1]
    bq = q.shape[0]
    qn = jnp.sum(q * q, axis=1, keepdims=True)
    cn = jnp.sum(cT * cT, axis=0, keepdims=True)
    d2 = qn - 2.0 * jnp.dot(q, cT, preferred_element_type=jnp.float32) + cn
    iota = jax.lax.broadcasted_iota(jnp.int32, (bq, nc), 1)
    wmat = jnp.zeros((bq, nc), jnp.float32)
    wsum = jnp.zeros((bq, 1), jnp.float32)
    for _ in range(3):
        mv = jnp.min(d2, axis=1, keepdims=True)
        sel = d2 == mv
        first = jnp.min(jnp.where(sel, iota, nc), axis=1, keepdims=True)
        onehot = iota == first
        w = 1.0 / jnp.maximum(mv, 1e-10)
        wmat = wmat + jnp.where(onehot, w, 0.0)
        wsum = wsum + w
        d2 = jnp.where(onehot, jnp.inf, d2)
    wmat = wmat / wsum
    o_ref[0] = jnp.dot(wmat, v_ref[0], preferred_element_type=jnp.float32)


def _knn3_pallas(qpos, cpos, vals, bq):
    Bb, nq, _ = qpos.shape
    nc = cpos.shape[1]
    C = vals.shape[2]
    cT = jnp.transpose(cpos, (0, 2, 1))
    return pl.pallas_call(
        _knn3_body,
        grid=(Bb, nq // bq),
        in_specs=[
            pl.BlockSpec((1, bq, 3), lambda b, i: (b, i, 0)),
            pl.BlockSpec((1, 3, nc), lambda b, i: (b, 0, 0)),
            pl.BlockSpec((1, nc, C), lambda b, i: (b, 0, 0)),
        ],
        out_specs=pl.BlockSpec((1, bq, C), lambda b, i: (b, i, 0)),
        out_shape=jax.ShapeDtypeStruct((Bb, nq, C), jnp.float32),
    )(qpos, cT, vals)


def _mlp_stack_body(nlin, relu_flags, softmax, maxpool, x_ref, *refs):
    o_ref = refs[-1]
    h = x_ref[0] if x_ref.shape[0] == 1 and len(x_ref.shape) == 3 else x_ref[...]
    for li in range(nlin):
        W = refs[2 * li][...]
        b = refs[2 * li + 1][...]
        h = jnp.dot(h, W, preferred_element_type=jnp.float32) + b
        if relu_flags[li]:
            h = jnp.maximum(h, 0.0)
    if softmax:
        mx = jnp.max(h, axis=-1, keepdims=True)
        h = (h - mx) - jnp.log(jnp.sum(jnp.exp(h - mx), axis=-1, keepdims=True))
    if maxpool:
        h = jnp.max(h, axis=0, keepdims=True)
    if len(o_ref.shape) == 3:
        o_ref[0] = h
    else:
        o_ref[...] = h


def _mlp_rows(x, layers, relu_flags=None, softmax=False, bm=2048):
    n, c = x.shape
    nlin = len(layers)
    if relu_flags is None:
        relu_flags = [True] * nlin
    ws = []
    for (W, b) in layers:
        ws.append(W)
        ws.append(b.reshape(1, -1))
    cout = layers[-1][0].shape[1]
    body = functools.partial(_mlp_stack_body, nlin, tuple(relu_flags),
                             softmax, False)
    return pl.pallas_call(
        body,
        grid=(n // bm,),
        in_specs=[pl.BlockSpec((bm, c), lambda i: (i, 0))]
        + [pl.BlockSpec(w.shape, lambda i: (0, 0)) for w in ws],
        out_specs=pl.BlockSpec((bm, cout), lambda i: (i, 0)),
        out_shape=jax.ShapeDtypeStruct((n, cout), jnp.float32),
    )(x, *ws)


def _mlp_batched(x, layers, maxpool=False):
    Bb, n, c = x.shape
    nlin = len(layers)
    ws = []
    for (W, b) in layers:
        ws.append(W)
        ws.append(b.reshape(1, -1))
    cout = layers[-1][0].shape[1]
    nout = 1 if maxpool else n
    body = functools.partial(_mlp_stack_body, nlin, tuple([True] * nlin),
                             False, maxpool)
    out = pl.pallas_call(
        body,
        grid=(Bb,),
        in_specs=[pl.BlockSpec((1, n, c), lambda b: (b, 0, 0))]
        + [pl.BlockSpec(w.shape, lambda b: (0, 0)) for w in ws],
        out_specs=pl.BlockSpec((1, nout, cout), lambda b: (b, 0, 0)),
        out_shape=jax.ShapeDtypeStruct((Bb, nout, cout), jnp.float32),
    )(x, *ws)
    return out[:, 0, :] if maxpool else out


def _ball_level(pos, centers, specs):
    d2 = jnp.sum((centers[:, :, None, :] - pos[:, None, :, :]) ** 2, axis=-1)
    kmax = max(K for (_, K) in specs)
    negd, sidx = jax.lax.approx_max_k(-d2, kmax, recall_target=1.0)
    sidx = sidx.astype(jnp.int32)
    sd2 = -negd
    outs = []
    for (r, K) in specs:
        idxK = sidx[..., :K]
        valid = sd2[..., :K] <= r * r
        outs.append(jnp.where(valid, idxK, idxK[..., :1]))
    return outs


def _g1_body(K, bm, r_ref, w1, b1, w2, b2, w3, b3, o_ref):
    h = r_ref[...]
    h = jnp.maximum(jnp.dot(h, w1[...], preferred_element_type=jnp.float32) + b1[...], 0.0)
    h = jnp.maximum(jnp.dot(h, w2[...], preferred_element_type=jnp.float32) + b2[...], 0.0)
    h = jnp.maximum(jnp.dot(h, w3[...], preferred_element_type=jnp.float32) + b3[...], 0.0)
    for i in range(bm):
        o_ref[i:i + 1, :] = jnp.max(h[i * K:(i + 1) * K, :], axis=0,
                                    keepdims=True)


def _grouped_mlp_rel(rel2d, K, bm, layers):
    R = rel2d.shape[0]
    c3 = layers[2][0].shape[1]
    nctr = R // K
    ws = []
    for (W, b) in layers:
        ws.append(W)
        ws.append(b.reshape(1, -1))
    body = functools.partial(_g1_body, K, bm)
    return pl.pallas_call(
        body,
        grid=(R // (bm * K),),
        in_specs=[pl.BlockSpec((bm * K, 3), lambda g: (g, 0))]
        + [pl.BlockSpec(w.shape, lambda g: (0, 0)) for w in ws],
        out_specs=pl.BlockSpec((bm, c3), lambda g: (g, 0)),
        out_shape=jax.ShapeDtypeStruct((nctr, c3), jnp.float32),
    )(rel2d, *ws)


def _g2_body(K, bm, n, a_ref, bc_ref, idx_ref, w2, b2, w3, b3, o_ref):
    A = a_ref[0]
    idxv = idx_ref[...]
    lane_n = jax.lax.broadcasted_iota(jnp.int32, (bm * K, n), 1)
    oh = jnp.where(idxv == lane_n, 1.0, 0.0)
    G = jnp.dot(oh, A, preferred_element_type=jnp.float32)
    gid = jax.lax.broadcasted_iota(jnp.int32, (bm * K, 1), 0) // K
    lane_b = jax.lax.broadcasted_iota(jnp.int32, (bm * K, bm), 1)
    ohg = jnp.where(gid == lane_b, 1.0, 0.0)
    Brep = jnp.dot(ohg, bc_ref[0], preferred_element_type=jnp.float32)
    h = jnp.maximum(G - Brep, 0.0)
    h = jnp.maximum(jnp.dot(h, w2[...], preferred_element_type=jnp.float32) + b2[...], 0.0)
    h = jnp.maximum(jnp.dot(h, w3[...], preferred_element_type=jnp.float32) + b3[...], 0.0)
    for i in range(bm):
        o_ref[0, i:i + 1, :] = jnp.max(h[i * K:(i + 1) * K, :], axis=0,
                                       keepdims=True)


def _grouped_mlp_gather(A, Bc2d, idxcol, K, bm, layers23):
    Bb, n, c1 = A.shape
    m = Bc2d.shape[0] // Bb
    c3 = layers23[1][0].shape[1]
    gpb = m // bm
    ws = []
    for (W, b) in layers23:
        ws.append(W)
        ws.append(b.reshape(1, -1))
    body = functools.partial(_g2_body, K, bm, n)
    ngrp = Bb * gpb
    bc3 = Bc2d.reshape(ngrp, bm, c1)
    out = pl.pallas_call(
        body,
        grid=(Bb, gpb),
        in_specs=[
            pl.BlockSpec((1, n, c1), lambda b, g: (b, 0, 0)),
            pl.BlockSpec((1, bm, c1), lambda b, g, _g=gpb: (b * _g + g, 0, 0)),
            pl.BlockSpec((bm * K, 1), lambda b, g, _g=gpb: (b * _g + g, 0)),
        ]
        + [pl.BlockSpec(w.shape, lambda b, g: (0, 0)) for w in ws],
        out_specs=pl.BlockSpec((1, bm, c3), lambda b, g, _g=gpb: (b * _g + g, 0, 0)),
        out_shape=jax.ShapeDtypeStruct((ngrp, bm, c3), jnp.float32),
    )(A, bc3, idxcol, *ws)
    return out.reshape(Bb * m, c3)


def _sa_msg1(pos, m, radii, Ks, mlps, bms):
    cidx = _fps_pallas(pos, m)
    centers = _gather_j(pos, cidx)
    idxs = _ball_level(pos, centers, list(zip(radii, Ks)))
    outs = []
    for idx, ps, K, bm in zip(idxs, mlps, Ks, bms):
        rel = _gather_j(pos, idx) - centers[:, :, None, :]
        rel2d = rel.reshape(_B * m * K, 3)
        o = _grouped_mlp_rel(rel2d, K, bm, ps)
        outs.append(o.reshape(_B, m, -1))
    return jnp.concatenate(outs, axis=-1), centers


def _sa_msg2(x, pos, m, radii, Ks, mlps, bms):
    cidx = _fps_pallas(pos, m)
    centers = _gather_j(pos, cidx)
    idxs = _ball_level(pos, centers, list(zip(radii, Ks)))
    n = pos.shape[1]
    xp = jnp.concatenate([x, pos], axis=-1).reshape(_B * n, -1)
    cflat = centers.reshape(_B * m, 3)
    outs = []
    for idx, ps, K, bm in zip(idxs, mlps, Ks, bms):
        (W1, b1), l2, l3 = ps
        A = _mlp_rows(xp, [(W1, b1)], relu_flags=[False],
                      bm=_B * n // 8).reshape(_B, n, -1)
        W1p = W1[-3:]
        Bc2d = _mlp_rows(cflat, [(W1p, jnp.zeros((W1.shape[1],), jnp.float32))],
                         relu_flags=[False], bm=_B * m // 8)
        idxcol = idx.reshape(_B * m * K, 1)
        o = _grouped_mlp_gather(A, Bc2d, idxcol, K, bm, [l2, l3])
        outs.append(o.reshape(_B, m, -1))
    return jnp.concatenate(outs, axis=-1), centers


def kernel(pos, batch, params):
    del batch
    pos3 = pos.reshape(_B, _NPTS, 3)
    x1, p1 = _sa_msg1(pos3, 512, (0.1, 0.2, 0.4), (16, 32, 128),
                      params['sa1'], bms=(64, 32, 16))
    x2, p2 = _sa_msg2(x1, p1, 128, (0.2, 0.4, 0.8), (32, 64, 128),
                      params['sa2'], bms=(16, 8, 4))
    g = _mlp_batched(jnp.concatenate([x2, p2], axis=-1), params['sa3'],
                     maxpool=True)
    gb = jnp.broadcast_to(g[:, None, :], (_B, 128, g.shape[1]))
    h3 = _mlp_batched(jnp.concatenate([gb, x2], axis=-1), params['fp3'])
    xi2 = _knn3_pallas(p1, p2, h3, bq=512)
    h2 = _mlp_batched(jnp.concatenate([xi2, x1], axis=-1), params['fp2'])
    xi1 = _knn3_pallas(pos3, p1, h2, bq=1024)
    head = list(params['fp1']) + [params['lin1'], params['lin2'], params['lin3']]
    out = _mlp_rows(xi1.reshape(_B * _NPTS, 128), head,
                    relu_flags=[True, True, True, True, False, False],
                    softmax=True, bm=2048)
    return out


# speedup vs baseline: 1.0992x; 1.0053x over previous
import functools

import jax
import jax.numpy as jnp
import numpy as np
from jax import lax
from jax.experimental import pallas as pl
from jax.experimental.pallas import tpu as pltpu

_B = 8
_NPTS = 4096
_NCLS = 13


def _gather_j(x, idx):
    bidx = jnp.arange(x.shape[0]).reshape((-1,) + (1,) * (idx.ndim - 1))
    return x[bidx, idx]


def _fps_body(m, n, px, py, pz, o_ref):
    x = px[...]
    y = py[...]
    z = pz[...]
    iota = jax.lax.broadcasted_iota(jnp.int32, (_B, n), 1)
    iom = jax.lax.broadcasted_iota(jnp.int32, (_B, m), 1)

    def step(t, carry):
        mind, last, acc = carry
        acc = acc + jnp.where(iom == t, 1, 0) * last
        sel = iota == last
        lx = jnp.sum(jnp.where(sel, x, 0.0), axis=1, keepdims=True)
        ly = jnp.sum(jnp.where(sel, y, 0.0), axis=1, keepdims=True)
        lz = jnp.sum(jnp.where(sel, z, 0.0), axis=1, keepdims=True)
        dx = x - lx
        dy = y - ly
        dz = z - lz
        d = dx * dx + dy * dy
        d = d + dz * dz
        mind = jnp.minimum(mind, d)
        mx = jnp.max(mind, axis=1, keepdims=True)
        cand = jnp.where(mind == mx, iota, n)
        nxt = jnp.min(cand, axis=1, keepdims=True)
        return mind, nxt, acc

    o_ref[...] = jnp.zeros((_B, m), jnp.int32)
    init = (jnp.full((_B, n), 1e10, jnp.float32),
            jnp.zeros((_B, 1), jnp.int32),
            o_ref[...])
    _, _, acc = jax.lax.fori_loop(0, m, step, init)
    o_ref[...] = acc


def _fps_pallas(pos, m):
    n = pos.shape[1]
    planes = jnp.transpose(pos, (2, 0, 1))
    px, py, pz = planes[0], planes[1], planes[2]
    body = functools.partial(_fps_body, m, n)
    return pl.pallas_call(
        body,
        grid=(1,),
        in_specs=[pl.BlockSpec((_B, n), lambda i: (0, 0))] * 3,
        out_specs=pl.BlockSpec((_B, m), lambda i: (0, 0)),
        out_shape=jax.ShapeDtypeStruct((_B, m), jnp.int32),
    )(px, py, pz)


def _knn3_body(q_ref, ct_ref, v_ref, o_ref):
    q = q_ref[0]
    cT = ct_ref[0]
    nc = cT.shape[1]
    bq = q.shape[0]
    qn = jnp.sum(q * q, axis=1, keepdims=True)
    cn = jnp.sum(cT * cT, axis=0, keepdims=True)
    d2 = qn - 2.0 * jnp.dot(q, cT, preferred_element_type=jnp.float32) + cn
    iota = jax.lax.broadcasted_iota(jnp.int32, (bq, nc), 1)
    wmat = jnp.zeros((bq, nc), jnp.float32)
    wsum = jnp.zeros((bq, 1), jnp.float32)
    for _ in range(3):
        mv = jnp.min(d2, axis=1, keepdims=True)
        sel = d2 == mv
        first = jnp.min(jnp.where(sel, iota, nc), axis=1, keepdims=True)
        onehot = iota == first
        w = 1.0 / jnp.maximum(mv, 1e-10)
        wmat = wmat + jnp.where(onehot, w, 0.0)
        wsum = wsum + w
        d2 = jnp.where(onehot, jnp.inf, d2)
    wmat = wmat / wsum
    o_ref[0] = jnp.dot(wmat, v_ref[0], preferred_element_type=jnp.float32)


def _knn3_pallas(qpos, cpos, vals, bq):
    Bb, nq, _ = qpos.shape
    nc = cpos.shape[1]
    C = vals.shape[2]
    cT = jnp.transpose(cpos, (0, 2, 1))
    return pl.pallas_call(
        _knn3_body,
        grid=(Bb, nq // bq),
        in_specs=[
            pl.BlockSpec((1, bq, 3), lambda b, i: (b, i, 0)),
            pl.BlockSpec((1, 3, nc), lambda b, i: (b, 0, 0)),
            pl.BlockSpec((1, nc, C), lambda b, i: (b, 0, 0)),
        ],
        out_specs=pl.BlockSpec((1, bq, C), lambda b, i: (b, i, 0)),
        out_shape=jax.ShapeDtypeStruct((Bb, nq, C), jnp.float32),
    )(qpos, cT, vals)


def _mlp_stack_body(nlin, relu_flags, softmax, maxpool, x_ref, *refs):
    o_ref = refs[-1]
    h = x_ref[0] if x_ref.shape[0] == 1 and len(x_ref.shape) == 3 else x_ref[...]
    for li in range(nlin):
        W = refs[2 * li][...]
        b = refs[2 * li + 1][...]
        h = jnp.dot(h, W, preferred_element_type=jnp.float32) + b
        if relu_flags[li]:
            h = jnp.maximum(h, 0.0)
    if softmax:
        mx = jnp.max(h, axis=-1, keepdims=True)
        h = (h - mx) - jnp.log(jnp.sum(jnp.exp(h - mx), axis=-1, keepdims=True))
    if maxpool:
        h = jnp.max(h, axis=0, keepdims=True)
    if len(o_ref.shape) == 3:
        o_ref[0] = h
    else:
        o_ref[...] = h


def _mlp_rows(x, layers, relu_flags=None, softmax=False, bm=2048):
    n, c = x.shape
    nlin = len(layers)
    if relu_flags is None:
        relu_flags = [True] * nlin
    ws = []
    for (W, b) in layers:
        ws.append(W)
        ws.append(b.reshape(1, -1))
    cout = layers[-1][0].shape[1]
    body = functools.partial(_mlp_stack_body, nlin, tuple(relu_flags),
                             softmax, False)
    return pl.pallas_call(
        body,
        grid=(n // bm,),
        in_specs=[pl.BlockSpec((bm, c), lambda i: (i, 0))]
        + [pl.BlockSpec(w.shape, lambda i: (0, 0)) for w in ws],
        out_specs=pl.BlockSpec((bm, cout), lambda i: (i, 0)),
        out_shape=jax.ShapeDtypeStruct((n, cout), jnp.float32),
    )(x, *ws)


def _mlp_batched(x, layers, maxpool=False):
    Bb, n, c = x.shape
    nlin = len(layers)
    ws = []
    for (W, b) in layers:
        ws.append(W)
        ws.append(b.reshape(1, -1))
    cout = layers[-1][0].shape[1]
    nout = 1 if maxpool else n
    body = functools.partial(_mlp_stack_body, nlin, tuple([True] * nlin),
                             False, maxpool)
    out = pl.pallas_call(
        body,
        grid=(Bb,),
        in_specs=[pl.BlockSpec((1, n, c), lambda b: (b, 0, 0))]
        + [pl.BlockSpec(w.shape, lambda b: (0, 0)) for w in ws],
        out_specs=pl.BlockSpec((1, nout, cout), lambda b: (b, 0, 0)),
        out_shape=jax.ShapeDtypeStruct((Bb, nout, cout), jnp.float32),
    )(x, *ws)
    return out[:, 0, :] if maxpool else out


def _ball_level(pos, centers, specs):
    d2 = jnp.sum((centers[:, :, None, :] - pos[:, None, :, :]) ** 2, axis=-1)
    kmax = max(K for (_, K) in specs)
    negd, sidx = jax.lax.approx_max_k(-d2, kmax, recall_target=1.0)
    sidx = sidx.astype(jnp.int32)
    sd2 = -negd
    outs = []
    for (r, K) in specs:
        idxK = sidx[..., :K]
        valid = sd2[..., :K] <= r * r
        outs.append(jnp.where(valid, idxK, idxK[..., :1]))
    return outs


def _g1_body(K, bm, r_ref, w1, b1, w2, b2, w3, b3, o_ref):
    h = r_ref[...]
    h = jnp.maximum(jnp.dot(h, w1[...], preferred_element_type=jnp.float32) + b1[...], 0.0)
    h = jnp.maximum(jnp.dot(h, w2[...], preferred_element_type=jnp.float32) + b2[...], 0.0)
    h = jnp.maximum(jnp.dot(h, w3[...], preferred_element_type=jnp.float32) + b3[...], 0.0)
    for i in range(bm):
        o_ref[i:i + 1, :] = jnp.max(h[i * K:(i + 1) * K, :], axis=0,
                                    keepdims=True)


def _grouped_mlp_rel(rel2d, K, bm, layers):
    R = rel2d.shape[0]
    c3 = layers[2][0].shape[1]
    nctr = R // K
    ws = []
    for (W, b) in layers:
        ws.append(W)
        ws.append(b.reshape(1, -1))
    body = functools.partial(_g1_body, K, bm)
    return pl.pallas_call(
        body,
        grid=(R // (bm * K),),
        in_specs=[pl.BlockSpec((bm * K, 3), lambda g: (g, 0))]
        + [pl.BlockSpec(w.shape, lambda g: (0, 0)) for w in ws],
        out_specs=pl.BlockSpec((bm, c3), lambda g: (g, 0)),
        out_shape=jax.ShapeDtypeStruct((nctr, c3), jnp.float32),
    )(rel2d, *ws)


def _g2_body(K, bm, n, a_ref, bc_ref, idx_ref, w2, b2, w3, b3, o_ref):
    A = a_ref[0]
    idxv = idx_ref[...]
    lane_n = jax.lax.broadcasted_iota(jnp.int32, (bm * K, n), 1)
    oh = jnp.where(idxv == lane_n, 1.0, 0.0)
    G = jnp.dot(oh, A, preferred_element_type=jnp.float32)
    gid = jax.lax.broadcasted_iota(jnp.int32, (bm * K, 1), 0) // K
    lane_b = jax.lax.broadcasted_iota(jnp.int32, (bm * K, bm), 1)
    ohg = jnp.where(gid == lane_b, 1.0, 0.0)
    Brep = jnp.dot(ohg, bc_ref[0], preferred_element_type=jnp.float32)
    h = jnp.maximum(G - Brep, 0.0)
    h = jnp.maximum(jnp.dot(h, w2[...], preferred_element_type=jnp.float32) + b2[...], 0.0)
    h = jnp.maximum(jnp.dot(h, w3[...], preferred_element_type=jnp.float32) + b3[...], 0.0)
    for i in range(bm):
        o_ref[0, i:i + 1, :] = jnp.max(h[i * K:(i + 1) * K, :], axis=0,
                                       keepdims=True)


def _grouped_mlp_gather(A, Bc2d, idxcol, K, bm, layers23):
    Bb, n, c1 = A.shape
    m = Bc2d.shape[0] // Bb
    c3 = layers23[1][0].shape[1]
    gpb = m // bm
    ws = []
    for (W, b) in layers23:
        ws.append(W)
        ws.append(b.reshape(1, -1))
    body = functools.partial(_g2_body, K, bm, n)
    ngrp = Bb * gpb
    bc3 = Bc2d.reshape(ngrp, bm, c1)
    out = pl.pallas_call(
        body,
        grid=(Bb, gpb),
        in_specs=[
            pl.BlockSpec((1, n, c1), lambda b, g: (b, 0, 0)),
            pl.BlockSpec((1, bm, c1), lambda b, g, _g=gpb: (b * _g + g, 0, 0)),
            pl.BlockSpec((bm * K, 1), lambda b, g, _g=gpb: (b * _g + g, 0)),
        ]
        + [pl.BlockSpec(w.shape, lambda b, g: (0, 0)) for w in ws],
        out_specs=pl.BlockSpec((1, bm, c3), lambda b, g, _g=gpb: (b * _g + g, 0, 0)),
        out_shape=jax.ShapeDtypeStruct((ngrp, bm, c3), jnp.float32),
    )(A, bc3, idxcol, *ws)
    return out.reshape(Bb * m, c3)


def _sa_msg1(pos, m, radii, Ks, mlps, bms):
    cidx = _fps_pallas(pos, m)
    centers = _gather_j(pos, cidx)
    idxs = _ball_level(pos, centers, list(zip(radii, Ks)))
    outs = []
    for idx, ps, K, bm in zip(idxs, mlps, Ks, bms):
        rel = _gather_j(pos, idx) - centers[:, :, None, :]
        rel2d = rel.reshape(_B * m * K, 3)
        o = _grouped_mlp_rel(rel2d, K, bm, ps)
        outs.append(o.reshape(_B, m, -1))
    return jnp.concatenate(outs, axis=-1), centers


def _sa_msg2(x, pos, m, radii, Ks, mlps, bms):
    cidx = _fps_pallas(pos, m)
    centers = _gather_j(pos, cidx)
    idxs = _ball_level(pos, centers, list(zip(radii, Ks)))
    n = pos.shape[1]
    xp = jnp.concatenate([x, pos], axis=-1).reshape(_B * n, -1)
    cflat = centers.reshape(_B * m, 3)
    outs = []
    for idx, ps, K, bm in zip(idxs, mlps, Ks, bms):
        (W1, b1), l2, l3 = ps
        A = _mlp_rows(xp, [(W1, b1)], relu_flags=[False],
                      bm=_B * n // 8).reshape(_B, n, -1)
        W1p = W1[-3:]
        Bc2d = _mlp_rows(cflat, [(W1p, jnp.zeros((W1.shape[1],), jnp.float32))],
                         relu_flags=[False], bm=_B * m // 8)
        idxcol = idx.reshape(_B * m * K, 1)
        o = _grouped_mlp_gather(A, Bc2d, idxcol, K, bm, [l2, l3])
        outs.append(o.reshape(_B, m, -1))
    return jnp.concatenate(outs, axis=-1), centers


def kernel(pos, batch, params):
    del batch
    pos3 = pos.reshape(_B, _NPTS, 3)
    x1, p1 = _sa_msg1(pos3, 512, (0.1, 0.2, 0.4), (16, 32, 128),
                      params['sa1'], bms=(64, 32, 32))
    x2, p2 = _sa_msg2(x1, p1, 128, (0.2, 0.4, 0.8), (32, 64, 128),
                      params['sa2'], bms=(16, 8, 4))
    g = _mlp_batched(jnp.concatenate([x2, p2], axis=-1), params['sa3'],
                     maxpool=True)
    gb = jnp.broadcast_to(g[:, None, :], (_B, 128, g.shape[1]))
    h3 = _mlp_batched(jnp.concatenate([gb, x2], axis=-1), params['fp3'])
    xi2 = _knn3_pallas(p1, p2, h3, bq=512)
    h2 = _mlp_batched(jnp.concatenate([xi2, x1], axis=-1), params['fp2'])
    xi1 = _knn3_pallas(pos3, p1, h2, bq=1024)
    head = list(params['fp1']) + [params['lin1'], params['lin2'], params['lin3']]
    out = _mlp_rows(xi1.reshape(_B * _NPTS, 128), head,
                    relu_flags=[True, True, True, True, False, False],
                    softmax=True, bm=4096)
    return out
